# Initial kernel scaffold; baseline (speedup 1.0000x reference)
import jax
import jax.numpy as jnp
from jax.experimental import pallas as pl


def kernel(query, key):
    raise NotImplementedError("write your pallas kernel here")


# speedup vs baseline: 111.9336x; 111.9336x over previous
import jax
import jax.numpy as jnp
from jax.experimental import pallas as pl
from jax.experimental.pallas import tpu as pltpu

_C = 64
_QH = 32
_KH = 48
_P = _KH - _QH + 1
_N = _C * _QH * _QH


def _patch_kernel(q_ref, ka_ref, kb_ref, idx_ref, val_ref, bv_ref, bi_ref):
    di = pl.program_id(0)

    @pl.when(di == 0)
    def _init():
        bv_ref[0] = jnp.float32(jnp.inf)
        bi_ref[0] = jnp.int32(0)

    q = q_ref[...]
    best_val = bv_ref[0]
    best_idx = bi_ref[0]
    for dj in range(_P):
        src = kb_ref if (dj % 2) else ka_ref
        off = (dj // 2) * 2 * _C
        w = src[0, :, off:off + _QH * _C]
        s = jnp.sum(jnp.abs(w - q))
        idx = di * _P + dj
        take = s < best_val
        best_val = jnp.where(take, s, best_val)
        best_idx = jnp.where(take, idx, best_idx)
    bv_ref[0] = best_val
    bi_ref[0] = best_idx

    @pl.when(di == _P - 1)
    def _fin():
        idx_ref[0] = bi_ref[0]
        val_ref[0, 0] = bv_ref[0] / jnp.float32(_N)


def kernel(query, key):
    P = int(key.shape[3]) - int(query.shape[3]) + 1

    q = query[0].transpose(1, 2, 0).reshape(_QH, _QH * _C)
    k3 = key[0].transpose(1, 2, 0)
    ka = k3.reshape(_KH, _KH * _C)
    kb = jnp.pad(k3[:, 1:, :], ((0, 0), (0, 1), (0, 0))).reshape(_KH, _KH * _C)
    rows = jnp.arange(_P)[:, None] + jnp.arange(_QH)[None, :]
    slabs_a = ka[rows]
    slabs_b = kb[rows]

    idx, val = pl.pallas_call(
        _patch_kernel,
        grid=(_P,),
        in_specs=(
            pl.BlockSpec((_QH, _QH * _C), lambda i: (0, 0)),
            pl.BlockSpec((1, _QH, _KH * _C), lambda i: (i, 0, 0)),
            pl.BlockSpec((1, _QH, _KH * _C), lambda i: (i, 0, 0)),
        ),
        out_specs=(
            pl.BlockSpec(memory_space=pltpu.SMEM),
            pl.BlockSpec(memory_space=pltpu.SMEM),
        ),
        out_shape=(
            jax.ShapeDtypeStruct((1,), jnp.int32),
            jax.ShapeDtypeStruct((1, 1), jnp.float32),
        ),
        scratch_shapes=(
            pltpu.SMEM((1,), jnp.float32),
            pltpu.SMEM((1,), jnp.int32),
        ),
    )(q, slabs_a, slabs_b)

    return (idx, P, val)


# speedup vs baseline: 144.6074x; 1.2919x over previous
import jax
import jax.numpy as jnp
from jax.experimental import pallas as pl
from jax.experimental.pallas import tpu as pltpu

_C = 64
_QH = 32
_KH = 48
_P = _KH - _QH + 1
_N = _C * _QH * _QH
_LW = _QH * _C
_KW = _KH * _C


def _patch_kernel(q_ref, ka_ref, kb_ref, idx_ref, val_ref):

    def make_a_body(r):
        def a_body(a, carry):
            best_val, best_idx = carry
            di = a * 8 + r
            accs = [jnp.zeros((8, 128), jnp.float32) for _ in range(_P)]
            for rb in range(4):
                base = pl.multiple_of((a + rb) * 8, 8)
                nrows = 8 if r == 0 else 16
                qb = q_ref[rb * 8:(rb + 1) * 8, :]
                sa = ka_ref[pl.ds(base, nrows), :]
                sb = kb_ref[pl.ds(base, nrows), :]
                if r:
                    sa = jax.lax.slice(sa, (r, 0), (r + 8, _KW))
                    sb = jax.lax.slice(sb, (r, 0), (r + 8, _KW))
                for dj in range(_P):
                    src = sb if (dj % 2) else sa
                    off = (dj // 2) * 128
                    w = jax.lax.slice(src, (0, off), (8, off + _LW))
                    d = jnp.abs(w - qb)
                    for c in range(_LW // 128):
                        accs[dj] = accs[dj] + jax.lax.slice(
                            d, (0, 128 * c), (8, 128 * (c + 1)))
            for dj in range(_P):
                s = jnp.sum(accs[dj])
                idx = di * _P + dj
                take = s < best_val
                best_val = jnp.where(take, s, best_val)
                best_idx = jnp.where(take, idx, best_idx)
            return best_val, best_idx
        return a_body

    carry = (jnp.float32(jnp.inf), jnp.int32(0))
    for r in range(8):
        n_a = 3 if r == 0 else 2
        carry = jax.lax.fori_loop(0, n_a, make_a_body(r), carry)
    best_val, best_idx = carry
    idx_ref[0] = best_idx
    val_ref[0, 0] = best_val / jnp.float32(_N)


def kernel(query, key):
    P = int(key.shape[3]) - int(query.shape[3]) + 1

    q = query[0].transpose(1, 2, 0).reshape(_QH, _LW)
    k3 = key[0].transpose(1, 2, 0)
    ka = k3.reshape(_KH, _KW)
    kb = jnp.pad(k3[:, 1:, :], ((0, 0), (0, 1), (0, 0))).reshape(_KH, _KW)

    idx, val = pl.pallas_call(
        _patch_kernel,
        out_shape=(
            jax.ShapeDtypeStruct((1,), jnp.int32),
            jax.ShapeDtypeStruct((1, 1), jnp.float32),
        ),
        out_specs=(
            pl.BlockSpec(memory_space=pltpu.SMEM),
            pl.BlockSpec(memory_space=pltpu.SMEM),
        ),
    )(q, ka, kb)

    return (idx, P, val)
